# Initial kernel scaffold; baseline (speedup 1.0000x reference)
import jax
import jax.numpy as jnp
from jax.experimental import pallas as pl


def kernel(x, edge_index, edge_attr, node_W, node_b, edge_W, edge_b, l0_W1, l0_b1, l0_W2, l0_b2, l1_W1, l1_b1, l1_W2, l1_b2, l2_W1, l2_b1, l2_W2, l2_b2, l3_W1, l3_b1, l3_W2, l3_b2, f_W1, f_b1, f_W2, f_b2):
    raise NotImplementedError("write your pallas kernel here")


# speedup vs baseline: 2.3903x; 2.3903x over previous
import jax
import jax.numpy as jnp
from jax import lax
from jax.experimental import pallas as pl
from jax.experimental.pallas import tpu as pltpu
from jax.experimental.pallas import tpu_sc as plsc

N = 10000
E = 320000
H = 64

NC = 2
NS = 16
NW = NC * NS

NP = 10240
C = 80
PER_W = E // NW
NCH = PER_W // C
NPAIR = NCH // 2
STRIPE = NP // NS


def _gather2_body(t1_hbm, t2_hbm, i1_hbm, i2_hbm, o1_hbm, o2_hbm,
                  i1_v, i2_v, r1_v, r2_v, sg, sw):
    wid = lax.axis_index("c") * NS + lax.axis_index("s")
    base = wid * PER_W
    pltpu.sync_copy(i1_hbm.at[wid], i1_v)
    pltpu.sync_copy(i2_hbm.at[wid], i2_v)

    def chunk(c0, c1):
        g = []
        for b, c in ((0, c0), (1, c1)):
            g.append(pltpu.async_copy(t1_hbm.at[i1_v.at[c]], r1_v.at[b], sg.at[b]))
            g.append(pltpu.async_copy(t2_hbm.at[i2_v.at[c]], r2_v.at[b], sg.at[b]))
        w = []
        for b, c in ((0, c0), (1, c1)):
            g[2 * b].wait()
            g[2 * b + 1].wait()
            row0 = base + c * C
            w.append(pltpu.async_copy(r1_v.at[b], o1_hbm.at[pl.ds(row0, C)], sw.at[b]))
            w.append(pltpu.async_copy(r2_v.at[b], o2_hbm.at[pl.ds(row0, C)], sw.at[b]))
        for d in w:
            d.wait()

    def pair(k2, _):
        chunk(2 * k2, 2 * k2 + 1)
        return ()

    lax.fori_loop(0, NPAIR, pair, (), unroll=False)
    g1 = pltpu.async_copy(t1_hbm.at[i1_v.at[NCH - 1]], r1_v.at[0], sg.at[0])
    g2 = pltpu.async_copy(t2_hbm.at[i2_v.at[NCH - 1]], r2_v.at[0], sg.at[0])
    g1.wait()
    g2.wait()
    row0 = base + (NCH - 1) * C
    pltpu.sync_copy(r1_v.at[0], o1_hbm.at[pl.ds(row0, C)])
    pltpu.sync_copy(r2_v.at[0], o2_hbm.at[pl.ds(row0, C)])


def _gather2(t1, t2, i1_3d, i2_3d):
    mesh = plsc.VectorSubcoreMesh(core_axis_name="c", subcore_axis_name="s")
    fn = pl.kernel(
        _gather2_body,
        out_type=(jax.ShapeDtypeStruct((E, H), jnp.float32),
                  jax.ShapeDtypeStruct((E, H), jnp.float32)),
        mesh=mesh,
        scratch_types=[
            pltpu.VMEM((NCH, C), jnp.int32),
            pltpu.VMEM((NCH, C), jnp.int32),
            pltpu.VMEM((2, C, H), jnp.float32),
            pltpu.VMEM((2, C, H), jnp.float32),
            pltpu.SemaphoreType.DMA((2,)),
            pltpu.SemaphoreType.DMA((2,)),
        ],
        compiler_params=pltpu.CompilerParams(use_tc_tiling_on_sc=False),
    )
    return fn(t1, t2, i1_3d, i2_3d)


def _scatter_body(q_hbm, idx_hbm, zero_hbm, out_hbm, i_v, q_v, acc, sl, ss):
    cid = lax.axis_index("c")
    sid = lax.axis_index("s")
    wid = cid * NS + sid
    r0 = sid * STRIPE
    pltpu.sync_copy(zero_hbm.at[pl.ds(r0, STRIPE)], acc.at[pl.ds(r0, STRIPE)])
    pltpu.sync_copy(idx_hbm.at[wid], i_v)
    plsc.subcore_barrier()

    ebase = wid * PER_W

    def chunk(c0, c1):
        l0 = pltpu.async_copy(q_hbm.at[pl.ds(ebase + c0 * C, C)], q_v.at[0], sl.at[0])
        l1 = pltpu.async_copy(q_hbm.at[pl.ds(ebase + c1 * C, C)], q_v.at[1], sl.at[1])
        l0.wait()
        s0 = pltpu.async_copy(q_v.at[0], acc.at[i_v.at[c0]], ss.at[0], add=True)
        l1.wait()
        s1 = pltpu.async_copy(q_v.at[1], acc.at[i_v.at[c1]], ss.at[1], add=True)
        s0.wait()
        s1.wait()

    def pair(k2, _):
        chunk(2 * k2, 2 * k2 + 1)
        return ()

    lax.fori_loop(0, NPAIR, pair, (), unroll=False)
    lt = pltpu.async_copy(q_hbm.at[pl.ds(ebase + (NCH - 1) * C, C)], q_v.at[0], sl.at[0])
    lt.wait()
    st = pltpu.async_copy(q_v.at[0], acc.at[i_v.at[NCH - 1]], ss.at[0], add=True)
    st.wait()

    plsc.subcore_barrier()
    pltpu.sync_copy(acc.at[pl.ds(r0, STRIPE)],
                    out_hbm.at[cid, pl.ds(r0, STRIPE)])


def _scatter_add(q, idx_3d, zeros_np):
    mesh = plsc.VectorSubcoreMesh(core_axis_name="c", subcore_axis_name="s")
    fn = pl.kernel(
        _scatter_body,
        out_type=jax.ShapeDtypeStruct((NC, NP, H), jnp.float32),
        mesh=mesh,
        scratch_types=[
            pltpu.VMEM((NCH, C), jnp.int32),
            pltpu.VMEM((2, C, H), jnp.float32),
            pltpu.VMEM_SHARED((NP, H), jnp.float32),
            pltpu.SemaphoreType.DMA((2,)),
            pltpu.SemaphoreType.DMA((2,)),
        ],
        compiler_params=pltpu.CompilerParams(use_tc_tiling_on_sc=False),
    )
    return fn(q, idx_3d, zeros_np)


BN = 1000
BP = 1280
BE = 4000


def _p0_body(x_ref, nW_ref, nb_ref, o_ref):
    o_ref[...] = (jnp.dot(x_ref[...], nW_ref[...],
                          preferred_element_type=jnp.float32) + nb_ref[...])


def _p0(x, node_W, node_b2d):
    return pl.pallas_call(
        _p0_body,
        grid=(N // BN,),
        in_specs=[
            pl.BlockSpec((BN, 128), lambda i: (i, 0)),
            pl.BlockSpec((128, H), lambda i: (0, 0)),
            pl.BlockSpec((1, H), lambda i: (0, 0)),
        ],
        out_specs=pl.BlockSpec((BN, H), lambda i: (i, 0)),
        out_shape=jax.ShapeDtypeStruct((N, H), jnp.float32),
    )(x, node_W, node_b2d)


def _padd_body(a_ref, b_ref, o_ref):
    o_ref[...] = a_ref[0] + b_ref[0]


def _padd(ab):
    return pl.pallas_call(
        _padd_body,
        grid=(NP // BP,),
        in_specs=[
            pl.BlockSpec((1, BP, H), lambda i: (0, i, 0)),
            pl.BlockSpec((1, BP, H), lambda i: (1, i, 0)),
        ],
        out_specs=pl.BlockSpec((BP, H), lambda i: (i, 0)),
        out_shape=jax.ShapeDtypeStruct((NP, H), jnp.float32),
    )(ab, ab)


def _edge_body(tA_ref, tB_ref, ea_ref, eW_ref, eb_ref, W1_ref,
               b1_ref, W2_ref, b2_ref, o_ref):
    ea = jnp.dot(ea_ref[...], eW_ref[...],
                 preferred_element_type=jnp.float32) + eb_ref[...]
    cat = jnp.concatenate([tA_ref[...] + tB_ref[...], ea], axis=1)
    pre = jnp.dot(cat, W1_ref[...],
                  preferred_element_type=jnp.float32) + b1_ref[...]
    m1 = jnp.maximum(pre, 0.0)
    o_ref[...] = jnp.dot(m1, W2_ref[...],
                         preferred_element_type=jnp.float32) + b2_ref[...]


def _edge_mlp(tA, tB, ea, eW, eb2d, W1, b1_2d, W2, b2_2d):
    wspec = lambda r, c: pl.BlockSpec((r, c), lambda i: (0, 0))
    return pl.pallas_call(
        _edge_body,
        grid=(E // BE,),
        in_specs=[
            pl.BlockSpec((BE, H), lambda i: (i, 0)),
            pl.BlockSpec((BE, H), lambda i: (i, 0)),
            pl.BlockSpec((BE, 16), lambda i: (i, 0)),
            wspec(16, H), wspec(1, H), wspec(2 * H, H),
            wspec(1, H), wspec(H, H), wspec(1, H),
        ],
        out_specs=pl.BlockSpec((BE, H), lambda i: (i, 0)),
        out_shape=jax.ShapeDtypeStruct((E, H), jnp.float32),
    )(tA, tB, ea, eW, eb2d, W1, b1_2d, W2, b2_2d)


def _final_body(tA_ref, tB_ref, fW1_ref, fb1_ref, fW2T_ref, fb2_ref, o_ref):
    er = jnp.concatenate([tA_ref[...], tB_ref[...]], axis=1)
    t = jnp.maximum(jnp.dot(er, fW1_ref[...],
                            preferred_element_type=jnp.float32) + fb1_ref[...], 0.0)
    o_ref[...] = jnp.sum(t * fW2T_ref[...], axis=1, keepdims=True) + fb2_ref[0, 0]


def _final(tA, tB, f_W1, fb1_2d, fW2T, fb2_2d):
    return pl.pallas_call(
        _final_body,
        grid=(E // BE,),
        in_specs=[
            pl.BlockSpec((BE, H), lambda i: (i, 0)),
            pl.BlockSpec((BE, H), lambda i: (i, 0)),
            pl.BlockSpec((2 * H, H), lambda i: (0, 0)),
            pl.BlockSpec((1, H), lambda i: (0, 0)),
            pl.BlockSpec((1, H), lambda i: (0, 0)),
            pl.BlockSpec((1, 1), lambda i: (0, 0)),
        ],
        out_specs=pl.BlockSpec((BE, 1), lambda i: (i, 0)),
        out_shape=jax.ShapeDtypeStruct((E, 1), jnp.float32),
    )(tA, tB, f_W1, fb1_2d, fW2T, fb2_2d)


def kernel(x, edge_index, edge_attr, node_W, node_b, edge_W, edge_b,
           l0_W1, l0_b1, l0_W2, l0_b2,
           l1_W1, l1_b1, l1_W2, l1_b2,
           l2_W1, l2_b1, l2_W2, l2_b2,
           l3_W1, l3_b1, l3_W2, l3_b2,
           f_W1, f_b1, f_W2, f_b2):
    src = edge_index[0].astype(jnp.int32)
    dst = edge_index[1].astype(jnp.int32)
    src_g = src.reshape(NW, NCH, C)
    dst_g = dst.reshape(NW, NCH, C)
    zeros_np = jnp.zeros((NP, H), jnp.float32)

    W1 = [l0_W1, l1_W1, l2_W1, l3_W1]
    b1 = [l0_b1, l1_b1, l2_b1, l3_b1]
    W2 = [l0_W2, l1_W2, l2_W2, l3_W2]
    b2 = [l0_b2, l1_b2, l2_b2, l3_b2]
    r2 = lambda v: v.reshape(1, -1)

    p = jnp.pad(_p0(x, node_W, r2(node_b)), ((0, NP - N), (0, 0)))
    for l in range(4):
        tA, tB = _gather2(p, p, src_g, dst_g)
        q = _edge_mlp(tA, tB, edge_attr, edge_W, r2(edge_b), W1[l],
                      r2(b1[l]), W2[l], r2(b2[l]))
        parts = _scatter_add(q, dst_g, zeros_np)
        p = _padd(parts)

    tA, tB = _gather2(p, p, src_g, dst_g)
    s = _final(tA, tB, f_W1, r2(f_b1), f_W2.reshape(1, H), f_b2.reshape(1, 1))
    return s.reshape(E)
